# baseline (speedup 1.0000x reference)
import functools

import jax
import jax.numpy as jnp
from jax import lax
from jax.experimental import pallas as pl
from jax.experimental.pallas import tpu as pltpu
from jax.experimental.pallas import tpu_sc as plsc

N_NODES = 10000
N_EDGES = 320000
D = 128
NC, NS, NW = 2, 16, 32
CH = 128
EPW = N_EDGES // NW
SEG_CPW = 80
DEC_EPW = 2 * N_EDGES // NW
DEC_CPW = 160
IDXB = 40
ACC_N = 10112
ACC_CH = ACC_N // CH
ACC_T = (ACC_CH + NS - 1) // NS
PAD_NODE = N_NODES + 16

_mesh = plsc.VectorSubcoreMesh(
    core_axis_name="c", subcore_axis_name="s", num_cores=NC, num_subcores=NS)


@functools.partial(
    pl.kernel,
    out_type=jax.ShapeDtypeStruct((NC, ACC_N, D), jnp.float32),
    mesh=_mesh,
    scratch_types=[
        pltpu.VMEM_SHARED((ACC_N, D), jnp.float32),
        pltpu.VMEM((IDXB, CH), jnp.int32),
        pltpu.VMEM((IDXB, CH), jnp.int32),
        pltpu.VMEM((CH, D), jnp.float32),
        pltpu.VMEM((CH, D), jnp.float32),
        pltpu.SemaphoreType.DMA,
        pltpu.SemaphoreType.DMA,
        pltpu.SemaphoreType.DMA,
        pltpu.SemaphoreType.DMA,
    ],
)
def _seg_kernel(feat, src3d, dst3d, aggp, acc, src_st, dst_st,
                rows0, rows1, g0, g1, s0, s1):
    zf = rows0
    c = lax.axis_index("c")
    s = lax.axis_index("s")
    w = c * NS + s
    z16 = jnp.zeros((16,), jnp.float32)

    def zr(i, _):
        for j in range(D // 16):
            zf[i, pl.ds(j * 16, 16)] = z16
        return 0
    lax.fori_loop(0, CH, zr, 0)

    for t in range(ACC_T):
        ck = s + NS * t

        @pl.when(ck < ACC_CH)
        def _():
            pltpu.sync_copy(zf, acc.at[pl.ds(ck * CH, CH)])
    plsc.subcore_barrier()

    bufs = (rows0, rows1)
    sems = (g0, g1)
    ssems = (s0, s1)

    def fire(lc, p):
        pltpu.async_copy(feat.at[src_st.at[lc]], bufs[p], sems[p])

    def wait(lc, p):
        pltpu.make_async_copy(feat.at[src_st.at[lc]], bufs[p], sems[p]).wait()

    def fire_s(lc, p):
        pltpu.async_copy(bufs[p], acc.at[dst_st.at[lc]], ssems[p], add=True)

    def wait_s(lc, p):
        pltpu.make_async_copy(bufs[p], acc.at[dst_st.at[lc]], ssems[p]).wait()

    for b in range(SEG_CPW // IDXB):
        pltpu.sync_copy(src3d.at[w, pl.ds(b * IDXB, IDXB)], src_st)
        pltpu.sync_copy(dst3d.at[w, pl.ds(b * IDXB, IDXB)], dst_st)
        fire(0, 0)
        fire(1, 1)

        def step(i, _):
            for p in range(2):
                lc = 2 * i + p
                wait(lc, p)
                fire_s(lc, p)
            for p in range(2):
                lc = 2 * i + p
                wait_s(lc, p)
                nxt = lc + 2

                @pl.when(nxt < IDXB)
                def _():
                    fire(nxt, p)
            return 0
        lax.fori_loop(0, IDXB // 2, step, 0)
    plsc.subcore_barrier()

    for t in range(ACC_T):
        ck = s + NS * t

        @pl.when(ck < ACC_CH)
        def _():
            pltpu.sync_copy(acc.at[pl.ds(ck * CH, CH)], zf)
            pltpu.sync_copy(zf, aggp.at[c, pl.ds(ck * CH, CH)])


@functools.partial(
    pl.kernel,
    out_type=jax.ShapeDtypeStruct((NC, ACC_N, D), jnp.float32),
    mesh=_mesh,
    scratch_types=[
        pltpu.VMEM_SHARED((ACC_N, D), jnp.float32),
        pltpu.VMEM((SEG_CPW, CH), jnp.int32),
        pltpu.VMEM((CH, D), jnp.float32),
    ],
)
def _cnt_kernel(dst3d, cntp, cacc, dst_st, buf):
    c = lax.axis_index("c")
    s = lax.axis_index("s")
    w = c * NS + s
    z16 = jnp.zeros((16,), jnp.float32)
    o16 = jnp.ones((16,), jnp.float32)

    def zr(i, _):
        for j in range(D // 16):
            buf[i, pl.ds(j * 16, 16)] = z16
        return 0
    lax.fori_loop(0, CH, zr, 0)

    for t in range(ACC_T):
        ck = s + NS * t

        @pl.when(ck < ACC_CH)
        def _():
            pltpu.sync_copy(buf, cacc.at[pl.ds(ck * CH, CH)])

    def onesfill(i, _):
        buf[i, pl.ds(0, 16)] = o16
        return 0
    lax.fori_loop(0, CH, onesfill, 0)
    plsc.subcore_barrier()

    pltpu.sync_copy(dst3d.at[w], dst_st)

    def step(ch, _):
        pltpu.sync_copy(buf, cacc.at[dst_st.at[ch]], add=True)
        return 0
    lax.fori_loop(0, SEG_CPW, step, 0)
    plsc.subcore_barrier()

    for t in range(ACC_T):
        ck = s + NS * t

        @pl.when(ck < ACC_CH)
        def _():
            pltpu.sync_copy(cacc.at[pl.ds(ck * CH, CH)], buf)
            pltpu.sync_copy(buf, cntp.at[c, pl.ds(ck * CH, CH)])


@functools.partial(
    pl.kernel,
    out_type=jax.ShapeDtypeStruct((NW * DEC_CPW * CH // 8, D), jnp.float32),
    mesh=_mesh,
    scratch_types=[
        pltpu.VMEM((DEC_CPW, CH), jnp.int32),
        pltpu.VMEM((DEC_CPW, CH), jnp.int32),
        pltpu.VMEM((CH, D), jnp.float32),
        pltpu.VMEM((CH, D), jnp.float32),
        pltpu.VMEM((CH, D), jnp.float32),
        pltpu.VMEM((CH, D), jnp.float32),
        pltpu.VMEM((CH // 8, D), jnp.float32),
        pltpu.VMEM((1, D), jnp.float32),
        pltpu.SemaphoreType.DMA,
        pltpu.SemaphoreType.DMA,
        pltpu.SemaphoreType.DMA,
        pltpu.SemaphoreType.DMA,
        pltpu.SemaphoreType.DMA,
        pltpu.SemaphoreType.DMA,
        pltpu.SemaphoreType.DMA,
        pltpu.SemaphoreType.DMA,
    ],
)
def _dec_kernel(Ah, Bh, u3d, v3d, w2h, out, u_st, v_st, r0, r1, r2, r3,
                outstage, w2s, a0, a1, a2, a3, b0, b1, b2, b3):
    c = lax.axis_index("c")
    s = lax.axis_index("s")
    w = c * NS + s
    rbase = w * (DEC_CPW * CH // 8)
    pltpu.sync_copy(u3d.at[w], u_st)
    pltpu.sync_copy(v3d.at[w], v_st)
    pltpu.sync_copy(w2h, w2s)
    w2c = [w2s[0, pl.ds(j * 16, 16)] for j in range(D // 16)]

    NP = 4
    R = (r0, r1, r2, r3)
    SA = (a0, a1, a2, a3)
    SB = (b0, b1, b2, b3)

    def fire_a(lc, p):
        pltpu.async_copy(Ah.at[u_st.at[lc]], R[p], SA[p])

    def wait_a(lc, p):
        pltpu.make_async_copy(Ah.at[u_st.at[lc]], R[p], SA[p]).wait()

    def fire_b(lc, p):
        pltpu.async_copy(Bh.at[v_st.at[lc]], R[p], SB[p], add=True)

    def wait_b(lc, p):
        pltpu.make_async_copy(Bh.at[v_st.at[lc]], R[p], SB[p]).wait()

    def compute(ch, p):
        r = R[p]

        @plsc.parallel_loop(0, CH // 8, unroll=2)
        def _(row):
            for k in range(8):
                e = row * 8 + k
                accv = None
                for j in range(D // 16):
                    t = jnp.maximum(r[e, pl.ds(j * 16, 16)], 0.0)
                    term = t * w2c[j]
                    accv = term if accv is None else accv + term
                outstage[row, pl.ds(k * 16, 16)] = accv
        pltpu.sync_copy(outstage, out.at[pl.ds(rbase + ch * (CH // 8), CH // 8)])

    for p in range(NP):
        fire_a(p, p)

    def step(i, _):
        for p in range(NP):
            lc = NP * i + p
            wait_a(lc, p)
            fire_b(lc, p)
        for p in range(NP):
            lc = NP * i + p
            wait_b(lc, p)
            compute(lc, p)
            nxt = lc + NP

            @pl.when(nxt < DEC_CPW)
            def _():
                fire_a(nxt, p)
        return 0
    lax.fori_loop(0, DEC_CPW // NP, step, 0)


_RED_BLK = 8192


def _tc_reduce(P, b2):
    def body(p_ref, b2_ref, o_ref):
        pv = p_ref[...]
        cols = [jnp.sum(pv[:, 16 * k:16 * (k + 1)], axis=1, keepdims=True)
                for k in range(8)]
        o_ref[...] = jnp.concatenate(cols, axis=1) + b2_ref[...]

    tot = P.shape[0]
    return pl.pallas_call(
        body,
        grid=(tot // _RED_BLK,),
        in_specs=[
            pl.BlockSpec((_RED_BLK, D), lambda i: (i, 0)),
            pl.BlockSpec((1, 8), lambda i: (0, 0)),
        ],
        out_specs=pl.BlockSpec((_RED_BLK, 8), lambda i: (i, 0)),
        out_shape=jax.ShapeDtypeStruct((tot, 8), jnp.float32),
    )(P, b2)

_ROWS_BLK = 1000


def _tc_layer1(x, aa, ab, ca, cb, wl, wr, b):
    def body(x_ref, aa_ref, ab_ref, ca_ref, cb_ref, wl_ref, wr_ref, b_ref, o_ref):
        cnt = ca_ref[:, 0:1] + cb_ref[:, 0:1]
        inv = 1.0 / jnp.maximum(cnt, 1.0)
        mean = (aa_ref[...] + ab_ref[...]) * inv
        acc = jnp.dot(mean, wl_ref[...], preferred_element_type=jnp.float32)
        acc = acc + jnp.dot(x_ref[...], wr_ref[...], preferred_element_type=jnp.float32)
        o_ref[...] = jnp.maximum(acc + b_ref[...], 0.0)

    return pl.pallas_call(
        body,
        grid=(N_NODES // _ROWS_BLK,),
        in_specs=[
            pl.BlockSpec((_ROWS_BLK, D), lambda i: (i, 0)),
            pl.BlockSpec((_ROWS_BLK, D), lambda i: (i, 0)),
            pl.BlockSpec((_ROWS_BLK, D), lambda i: (i, 0)),
            pl.BlockSpec((_ROWS_BLK, D), lambda i: (i, 0)),
            pl.BlockSpec((_ROWS_BLK, D), lambda i: (i, 0)),
            pl.BlockSpec((D, D), lambda i: (0, 0)),
            pl.BlockSpec((D, D), lambda i: (0, 0)),
            pl.BlockSpec((1, D), lambda i: (0, 0)),
        ],
        out_specs=pl.BlockSpec((_ROWS_BLK, D), lambda i: (i, 0)),
        out_shape=jax.ShapeDtypeStruct((N_NODES, D), jnp.float32),
    )(x, aa, ab, ca, cb, wl, wr, b)


def _tc_layer2(h, aa, ab, ca, cb, wl, wr, bl, wa, wb, bd):
    def body(h_ref, aa_ref, ab_ref, ca_ref, cb_ref, wl_ref, wr_ref, bl_ref,
             wa_ref, wb_ref, bd_ref, a_ref, b_ref):
        cnt = ca_ref[:, 0:1] + cb_ref[:, 0:1]
        inv = 1.0 / jnp.maximum(cnt, 1.0)
        mean = (aa_ref[...] + ab_ref[...]) * inv
        z = jnp.dot(mean, wl_ref[...], preferred_element_type=jnp.float32)
        z = z + jnp.dot(h_ref[...], wr_ref[...], preferred_element_type=jnp.float32)
        z = z + bl_ref[...]
        a_ref[...] = jnp.dot(z, wa_ref[...], preferred_element_type=jnp.float32) + bd_ref[...]
        b_ref[...] = jnp.dot(z, wb_ref[...], preferred_element_type=jnp.float32)

    return pl.pallas_call(
        body,
        grid=(N_NODES // _ROWS_BLK,),
        in_specs=[
            pl.BlockSpec((_ROWS_BLK, D), lambda i: (i, 0)),
            pl.BlockSpec((_ROWS_BLK, D), lambda i: (i, 0)),
            pl.BlockSpec((_ROWS_BLK, D), lambda i: (i, 0)),
            pl.BlockSpec((_ROWS_BLK, D), lambda i: (i, 0)),
            pl.BlockSpec((_ROWS_BLK, D), lambda i: (i, 0)),
            pl.BlockSpec((D, D), lambda i: (0, 0)),
            pl.BlockSpec((D, D), lambda i: (0, 0)),
            pl.BlockSpec((1, D), lambda i: (0, 0)),
            pl.BlockSpec((D, D), lambda i: (0, 0)),
            pl.BlockSpec((D, D), lambda i: (0, 0)),
            pl.BlockSpec((1, D), lambda i: (0, 0)),
        ],
        out_specs=[
            pl.BlockSpec((_ROWS_BLK, D), lambda i: (i, 0)),
            pl.BlockSpec((_ROWS_BLK, D), lambda i: (i, 0)),
        ],
        out_shape=[
            jax.ShapeDtypeStruct((N_NODES, D), jnp.float32),
            jax.ShapeDtypeStruct((N_NODES, D), jnp.float32),
        ],
    )(h, aa, ab, ca, cb, wl, wr, bl, wa, wb, bd)


def _pad_worker_idx(idx, epw, cpw, fill):
    idx = idx.astype(jnp.int32).reshape(NW, epw)
    pad = jnp.full((NW, cpw * CH - epw), fill, jnp.int32)
    return jnp.concatenate([idx, pad], axis=1).reshape(NW, cpw, CH)


def kernel(x, edge_index, pos_edge, neg_edge, Wl1, bl1, Wr1, Wl2, bl2, Wr2,
           Wd1, bd1, Wd2, bd2):
    src3d = _pad_worker_idx(edge_index[0], EPW, SEG_CPW, 0)
    dst3d = _pad_worker_idx(edge_index[1], EPW, SEG_CPW, PAD_NODE)
    u3d = _pad_worker_idx(jnp.concatenate([pos_edge[0], neg_edge[0]]),
                          DEC_EPW, DEC_CPW, 0)
    v3d = _pad_worker_idx(jnp.concatenate([pos_edge[1], neg_edge[1]]),
                          DEC_EPW, DEC_CPW, 0)

    cntp = _cnt_kernel(dst3d)
    ca = cntp[0, :N_NODES]
    cb = cntp[1, :N_NODES]
    aggp1 = _seg_kernel(x, src3d, dst3d)
    h = _tc_layer1(x, aggp1[0, :N_NODES], aggp1[1, :N_NODES], ca, cb,
                   Wl1.T, Wr1.T, bl1[None, :])
    aggp2 = _seg_kernel(h, src3d, dst3d)
    A, B = _tc_layer2(h, aggp2[0, :N_NODES], aggp2[1, :N_NODES], ca, cb,
                      Wl2.T, Wr2.T, bl2[None, :],
                      Wd1[:, :D].T, Wd1[:, D:].T, bd1[None, :])
    w2b = Wd2.reshape(1, D)
    lane_p = _dec_kernel(A, B, u3d, v3d, w2b)
    b2b = jnp.full((1, 8), bd2[0], jnp.float32)
    out = _tc_reduce(lane_p, b2b)
    return out.reshape(NW, DEC_CPW * CH)[:, :DEC_EPW].reshape(-1)


# speedup vs baseline: 1.2129x; 1.2129x over previous
import functools

import jax
import jax.numpy as jnp
from jax import lax
from jax.experimental import pallas as pl
from jax.experimental.pallas import tpu as pltpu
from jax.experimental.pallas import tpu_sc as plsc

N_NODES = 10000
N_EDGES = 320000
D = 128
NC, NS, NW = 2, 16, 32
CH = 128
EPW = N_EDGES // NW
SEG_CPW = 80
DEC_EPW = 2 * N_EDGES // NW
DEC_CPW = 160
IDXB = 40
ACC_N = 10112
ACC_CH = ACC_N // CH
ACC_T = (ACC_CH + NS - 1) // NS
PAD_NODE = N_NODES + 16

_mesh = plsc.VectorSubcoreMesh(
    core_axis_name="c", subcore_axis_name="s", num_cores=NC, num_subcores=NS)


@functools.partial(
    pl.kernel,
    out_type=jax.ShapeDtypeStruct((NC, ACC_N, D), jnp.float32),
    mesh=_mesh,
    scratch_types=[
        pltpu.VMEM_SHARED((ACC_N, D), jnp.float32),
        pltpu.VMEM((IDXB, CH), jnp.int32),
        pltpu.VMEM((IDXB, CH), jnp.int32),
        pltpu.VMEM((CH, D), jnp.float32),
        pltpu.VMEM((CH, D), jnp.float32),
        pltpu.SemaphoreType.DMA,
        pltpu.SemaphoreType.DMA,
    ],
)
def _seg_kernel(feat, src3d, dst3d, aggp, acc, src_st, dst_st,
                rows0, rows1, g0, g1):
    zf = rows0
    c = lax.axis_index("c")
    s = lax.axis_index("s")
    w = c * NS + s
    z16 = jnp.zeros((16,), jnp.float32)

    def zr(i, _):
        for j in range(D // 16):
            zf[i, pl.ds(j * 16, 16)] = z16
        return 0
    lax.fori_loop(0, CH, zr, 0)

    for t in range(ACC_T):
        ck = s + NS * t

        @pl.when(ck < ACC_CH)
        def _():
            pltpu.sync_copy(zf, acc.at[pl.ds(ck * CH, CH)])
    plsc.subcore_barrier()

    bufs = (rows0, rows1)
    sems = (g0, g1)

    def fire(lc, p):
        pltpu.async_copy(feat.at[src_st.at[lc]], bufs[p], sems[p])

    def wait(lc, p):
        pltpu.make_async_copy(feat.at[src_st.at[lc]], bufs[p], sems[p]).wait()

    def consume(lc, p):
        pltpu.sync_copy(bufs[p], acc.at[dst_st.at[lc]], add=True)

    for b in range(SEG_CPW // IDXB):
        pltpu.sync_copy(src3d.at[w, pl.ds(b * IDXB, IDXB)], src_st)
        pltpu.sync_copy(dst3d.at[w, pl.ds(b * IDXB, IDXB)], dst_st)
        fire(0, 0)
        fire(1, 1)

        def step(i, _):
            for p in range(2):
                lc = 2 * i + p
                wait(lc, p)
                consume(lc, p)
                nxt = lc + 2

                @pl.when(nxt < IDXB)
                def _():
                    fire(nxt, p)
            return 0
        lax.fori_loop(0, IDXB // 2, step, 0)
    plsc.subcore_barrier()

    for t in range(ACC_T):
        ck = s + NS * t

        @pl.when(ck < ACC_CH)
        def _():
            pltpu.sync_copy(acc.at[pl.ds(ck * CH, CH)], zf)
            pltpu.sync_copy(zf, aggp.at[c, pl.ds(ck * CH, CH)])


@functools.partial(
    pl.kernel,
    out_type=jax.ShapeDtypeStruct((NC, ACC_N, D), jnp.float32),
    mesh=_mesh,
    scratch_types=[
        pltpu.VMEM_SHARED((ACC_N, D), jnp.float32),
        pltpu.VMEM((SEG_CPW, CH), jnp.int32),
        pltpu.VMEM((CH, D), jnp.float32),
    ],
)
def _cnt_kernel(dst3d, cntp, cacc, dst_st, buf):
    c = lax.axis_index("c")
    s = lax.axis_index("s")
    w = c * NS + s
    z16 = jnp.zeros((16,), jnp.float32)
    o16 = jnp.ones((16,), jnp.float32)

    def zr(i, _):
        for j in range(D // 16):
            buf[i, pl.ds(j * 16, 16)] = z16
        return 0
    lax.fori_loop(0, CH, zr, 0)

    for t in range(ACC_T):
        ck = s + NS * t

        @pl.when(ck < ACC_CH)
        def _():
            pltpu.sync_copy(buf, cacc.at[pl.ds(ck * CH, CH)])

    def onesfill(i, _):
        buf[i, pl.ds(0, 16)] = o16
        return 0
    lax.fori_loop(0, CH, onesfill, 0)
    plsc.subcore_barrier()

    pltpu.sync_copy(dst3d.at[w], dst_st)

    def step(ch, _):
        pltpu.sync_copy(buf, cacc.at[dst_st.at[ch]], add=True)
        return 0
    lax.fori_loop(0, SEG_CPW, step, 0)
    plsc.subcore_barrier()

    for t in range(ACC_T):
        ck = s + NS * t

        @pl.when(ck < ACC_CH)
        def _():
            pltpu.sync_copy(cacc.at[pl.ds(ck * CH, CH)], buf)
            pltpu.sync_copy(buf, cntp.at[c, pl.ds(ck * CH, CH)])


@functools.partial(
    pl.kernel,
    out_type=jax.ShapeDtypeStruct((NW * DEC_CPW * CH // 8, D), jnp.float32),
    mesh=_mesh,
    scratch_types=[
        pltpu.VMEM((DEC_CPW, CH), jnp.int32),
        pltpu.VMEM((DEC_CPW, CH), jnp.int32),
        pltpu.VMEM((CH, D), jnp.float32),
        pltpu.VMEM((CH, D), jnp.float32),
        pltpu.VMEM((CH, D), jnp.float32),
        pltpu.VMEM((CH, D), jnp.float32),
        pltpu.VMEM((CH // 8, D), jnp.float32),
        pltpu.VMEM((1, D), jnp.float32),
        pltpu.SemaphoreType.DMA,
        pltpu.SemaphoreType.DMA,
        pltpu.SemaphoreType.DMA,
        pltpu.SemaphoreType.DMA,
    ],
)
def _dec_kernel(Ah, Bh, u3d, v3d, w2h, out, u_st, v_st, ra0, rb0, ra1, rb1,
                outstage, w2s, a0, b0, a1, b1):
    c = lax.axis_index("c")
    s = lax.axis_index("s")
    w = c * NS + s
    rbase = w * (DEC_CPW * CH // 8)
    pltpu.sync_copy(u3d.at[w], u_st)
    pltpu.sync_copy(v3d.at[w], v_st)
    pltpu.sync_copy(w2h, w2s)
    w2c = [w2s[0, pl.ds(j * 16, 16)] for j in range(D // 16)]

    RA = (ra0, ra1)
    RB = (rb0, rb1)
    SA = (a0, a1)
    SB = (b0, b1)

    def fire(lc, p):
        pltpu.async_copy(Ah.at[u_st.at[lc]], RA[p], SA[p])
        pltpu.async_copy(Bh.at[v_st.at[lc]], RB[p], SB[p])

    def wait(lc, p):
        pltpu.make_async_copy(Ah.at[u_st.at[lc]], RA[p], SA[p]).wait()
        pltpu.make_async_copy(Bh.at[v_st.at[lc]], RB[p], SB[p]).wait()

    def compute(ch, p):
        ra, rb = RA[p], RB[p]

        @plsc.parallel_loop(0, CH // 8, unroll=2)
        def _(row):
            for k in range(8):
                e = row * 8 + k
                accv = None
                for j in range(D // 16):
                    t = jnp.maximum(ra[e, pl.ds(j * 16, 16)] + rb[e, pl.ds(j * 16, 16)], 0.0)
                    term = t * w2c[j]
                    accv = term if accv is None else accv + term
                outstage[row, pl.ds(k * 16, 16)] = accv
        pltpu.sync_copy(outstage, out.at[pl.ds(rbase + ch * (CH // 8), CH // 8)])

    fire(0, 0)
    fire(1, 1)

    def step(i, _):
        for p in range(2):
            lc = 2 * i + p
            wait(lc, p)
            compute(lc, p)
            nxt = lc + 2

            @pl.when(nxt < DEC_CPW)
            def _():
                fire(nxt, p)
        return 0
    lax.fori_loop(0, DEC_CPW // 2, step, 0)


_RED_BLK = 8192


def _tc_reduce(P, b2):
    def body(p_ref, b2_ref, o_ref):
        pv = p_ref[...]
        cols = [jnp.sum(pv[:, 16 * k:16 * (k + 1)], axis=1, keepdims=True)
                for k in range(8)]
        o_ref[...] = jnp.concatenate(cols, axis=1) + b2_ref[...]

    tot = P.shape[0]
    return pl.pallas_call(
        body,
        grid=(tot // _RED_BLK,),
        in_specs=[
            pl.BlockSpec((_RED_BLK, D), lambda i: (i, 0)),
            pl.BlockSpec((1, 8), lambda i: (0, 0)),
        ],
        out_specs=pl.BlockSpec((_RED_BLK, 8), lambda i: (i, 0)),
        out_shape=jax.ShapeDtypeStruct((tot, 8), jnp.float32),
    )(P, b2)

_ROWS_BLK = 1000


def _tc_layer1(x, aa, ab, ca, cb, wl, wr, b):
    def body(x_ref, aa_ref, ab_ref, ca_ref, cb_ref, wl_ref, wr_ref, b_ref, o_ref):
        cnt = ca_ref[:, 0:1] + cb_ref[:, 0:1]
        inv = 1.0 / jnp.maximum(cnt, 1.0)
        mean = (aa_ref[...] + ab_ref[...]) * inv
        acc = jnp.dot(mean, wl_ref[...], preferred_element_type=jnp.float32)
        acc = acc + jnp.dot(x_ref[...], wr_ref[...], preferred_element_type=jnp.float32)
        o_ref[...] = jnp.maximum(acc + b_ref[...], 0.0)

    return pl.pallas_call(
        body,
        grid=(N_NODES // _ROWS_BLK,),
        in_specs=[
            pl.BlockSpec((_ROWS_BLK, D), lambda i: (i, 0)),
            pl.BlockSpec((_ROWS_BLK, D), lambda i: (i, 0)),
            pl.BlockSpec((_ROWS_BLK, D), lambda i: (i, 0)),
            pl.BlockSpec((_ROWS_BLK, D), lambda i: (i, 0)),
            pl.BlockSpec((_ROWS_BLK, D), lambda i: (i, 0)),
            pl.BlockSpec((D, D), lambda i: (0, 0)),
            pl.BlockSpec((D, D), lambda i: (0, 0)),
            pl.BlockSpec((1, D), lambda i: (0, 0)),
        ],
        out_specs=pl.BlockSpec((_ROWS_BLK, D), lambda i: (i, 0)),
        out_shape=jax.ShapeDtypeStruct((N_NODES, D), jnp.float32),
    )(x, aa, ab, ca, cb, wl, wr, b)


def _tc_layer2(h, aa, ab, ca, cb, wl, wr, bl, wa, wb, bd):
    def body(h_ref, aa_ref, ab_ref, ca_ref, cb_ref, wl_ref, wr_ref, bl_ref,
             wa_ref, wb_ref, bd_ref, a_ref, b_ref):
        cnt = ca_ref[:, 0:1] + cb_ref[:, 0:1]
        inv = 1.0 / jnp.maximum(cnt, 1.0)
        mean = (aa_ref[...] + ab_ref[...]) * inv
        z = jnp.dot(mean, wl_ref[...], preferred_element_type=jnp.float32)
        z = z + jnp.dot(h_ref[...], wr_ref[...], preferred_element_type=jnp.float32)
        z = z + bl_ref[...]
        a_ref[...] = jnp.dot(z, wa_ref[...], preferred_element_type=jnp.float32) + bd_ref[...]
        b_ref[...] = jnp.dot(z, wb_ref[...], preferred_element_type=jnp.float32)

    return pl.pallas_call(
        body,
        grid=(N_NODES // _ROWS_BLK,),
        in_specs=[
            pl.BlockSpec((_ROWS_BLK, D), lambda i: (i, 0)),
            pl.BlockSpec((_ROWS_BLK, D), lambda i: (i, 0)),
            pl.BlockSpec((_ROWS_BLK, D), lambda i: (i, 0)),
            pl.BlockSpec((_ROWS_BLK, D), lambda i: (i, 0)),
            pl.BlockSpec((_ROWS_BLK, D), lambda i: (i, 0)),
            pl.BlockSpec((D, D), lambda i: (0, 0)),
            pl.BlockSpec((D, D), lambda i: (0, 0)),
            pl.BlockSpec((1, D), lambda i: (0, 0)),
            pl.BlockSpec((D, D), lambda i: (0, 0)),
            pl.BlockSpec((D, D), lambda i: (0, 0)),
            pl.BlockSpec((1, D), lambda i: (0, 0)),
        ],
        out_specs=[
            pl.BlockSpec((_ROWS_BLK, D), lambda i: (i, 0)),
            pl.BlockSpec((_ROWS_BLK, D), lambda i: (i, 0)),
        ],
        out_shape=[
            jax.ShapeDtypeStruct((N_NODES, D), jnp.float32),
            jax.ShapeDtypeStruct((N_NODES, D), jnp.float32),
        ],
    )(h, aa, ab, ca, cb, wl, wr, bl, wa, wb, bd)


def _pad_worker_idx(idx, epw, cpw, fill):
    idx = idx.astype(jnp.int32).reshape(NW, epw)
    pad = jnp.full((NW, cpw * CH - epw), fill, jnp.int32)
    return jnp.concatenate([idx, pad], axis=1).reshape(NW, cpw, CH)


def kernel(x, edge_index, pos_edge, neg_edge, Wl1, bl1, Wr1, Wl2, bl2, Wr2,
           Wd1, bd1, Wd2, bd2):
    src3d = _pad_worker_idx(edge_index[0], EPW, SEG_CPW, 0)
    dst3d = _pad_worker_idx(edge_index[1], EPW, SEG_CPW, PAD_NODE)
    u3d = _pad_worker_idx(jnp.concatenate([pos_edge[0], neg_edge[0]]),
                          DEC_EPW, DEC_CPW, 0)
    v3d = _pad_worker_idx(jnp.concatenate([pos_edge[1], neg_edge[1]]),
                          DEC_EPW, DEC_CPW, 0)

    cntp = _cnt_kernel(dst3d)
    ca = cntp[0, :N_NODES]
    cb = cntp[1, :N_NODES]
    aggp1 = _seg_kernel(x, src3d, dst3d)
    h = _tc_layer1(x, aggp1[0, :N_NODES], aggp1[1, :N_NODES], ca, cb,
                   Wl1.T, Wr1.T, bl1[None, :])
    aggp2 = _seg_kernel(h, src3d, dst3d)
    A, B = _tc_layer2(h, aggp2[0, :N_NODES], aggp2[1, :N_NODES], ca, cb,
                      Wl2.T, Wr2.T, bl2[None, :],
                      Wd1[:, :D].T, Wd1[:, D:].T, bd1[None, :])
    w2b = Wd2.reshape(1, D)
    lane_p = _dec_kernel(A, B, u3d, v3d, w2b)
    b2b = jnp.full((1, 8), bd2[0], jnp.float32)
    out = _tc_reduce(lane_p, b2b)
    return out.reshape(NW, DEC_CPW * CH)[:, :DEC_EPW].reshape(-1)


# speedup vs baseline: 1.2154x; 1.0021x over previous
import functools

import jax
import jax.numpy as jnp
from jax import lax
from jax.experimental import pallas as pl
from jax.experimental.pallas import tpu as pltpu
from jax.experimental.pallas import tpu_sc as plsc

N_NODES = 10000
N_EDGES = 320000
D = 128
NC, NS, NW = 2, 16, 32
CH = 128
EPW = N_EDGES // NW
SEG_CPW = 80
DEC_EPW = 2 * N_EDGES // NW
DEC_CPW = 160
IDXB = 40
ACC_N = 10112
ACC_CH = ACC_N // CH
ACC_T = (ACC_CH + NS - 1) // NS
PAD_NODE = N_NODES + 16

_mesh = plsc.VectorSubcoreMesh(
    core_axis_name="c", subcore_axis_name="s", num_cores=NC, num_subcores=NS)


@functools.partial(
    pl.kernel,
    out_type=jax.ShapeDtypeStruct((NC, ACC_N, D), jnp.float32),
    mesh=_mesh,
    scratch_types=[
        pltpu.VMEM_SHARED((ACC_N, D), jnp.float32),
        pltpu.VMEM((IDXB, CH), jnp.int32),
        pltpu.VMEM((IDXB, CH), jnp.int32),
        pltpu.VMEM((CH, D), jnp.float32),
        pltpu.VMEM((CH, D), jnp.float32),
        pltpu.SemaphoreType.DMA,
        pltpu.SemaphoreType.DMA,
    ],
)
def _seg_kernel(feat, src3d, dst3d, aggp, acc, src_st, dst_st,
                rows0, rows1, g0, g1):
    zf = rows0
    c = lax.axis_index("c")
    s = lax.axis_index("s")
    w = c * NS + s
    z16 = jnp.zeros((16,), jnp.float32)

    def zr(i, _):
        for j in range(D // 16):
            zf[i, pl.ds(j * 16, 16)] = z16
        return 0
    lax.fori_loop(0, CH, zr, 0)

    for t in range(ACC_T):
        ck = s + NS * t

        @pl.when(ck < ACC_CH)
        def _():
            pltpu.sync_copy(zf, acc.at[pl.ds(ck * CH, CH)])
    plsc.subcore_barrier()

    bufs = (rows0, rows1)
    sems = (g0, g1)

    def fire(lc, p):
        pltpu.async_copy(feat.at[src_st.at[lc]], bufs[p], sems[p])

    def wait(lc, p):
        pltpu.make_async_copy(feat.at[src_st.at[lc]], bufs[p], sems[p]).wait()

    def consume(lc, p):
        pltpu.sync_copy(bufs[p], acc.at[dst_st.at[lc]], add=True)

    for b in range(SEG_CPW // IDXB):
        pltpu.sync_copy(src3d.at[w, pl.ds(b * IDXB, IDXB)], src_st)
        pltpu.sync_copy(dst3d.at[w, pl.ds(b * IDXB, IDXB)], dst_st)
        fire(0, 0)
        fire(1, 1)

        def step(i, _):
            for p in range(2):
                lc = 2 * i + p
                wait(lc, p)
                consume(lc, p)
                nxt = lc + 2

                @pl.when(nxt < IDXB)
                def _():
                    fire(nxt, p)
            return 0
        lax.fori_loop(0, IDXB // 2, step, 0)
    plsc.subcore_barrier()

    for t in range(ACC_T):
        ck = s + NS * t

        @pl.when(ck < ACC_CH)
        def _():
            pltpu.sync_copy(acc.at[pl.ds(ck * CH, CH)], zf)
            pltpu.sync_copy(zf, aggp.at[c, pl.ds(ck * CH, CH)])


@functools.partial(
    pl.kernel,
    out_type=jax.ShapeDtypeStruct((NC, ACC_N, D), jnp.float32),
    mesh=_mesh,
    scratch_types=[
        pltpu.VMEM_SHARED((ACC_N, D), jnp.float32),
        pltpu.VMEM((SEG_CPW, CH), jnp.int32),
        pltpu.VMEM((CH, D), jnp.float32),
    ],
)
def _cnt_kernel(dst3d, cntp, cacc, dst_st, buf):
    c = lax.axis_index("c")
    s = lax.axis_index("s")
    w = c * NS + s
    z16 = jnp.zeros((16,), jnp.float32)
    o16 = jnp.ones((16,), jnp.float32)

    def zr(i, _):
        for j in range(D // 16):
            buf[i, pl.ds(j * 16, 16)] = z16
        return 0
    lax.fori_loop(0, CH, zr, 0)

    for t in range(ACC_T):
        ck = s + NS * t

        @pl.when(ck < ACC_CH)
        def _():
            pltpu.sync_copy(buf, cacc.at[pl.ds(ck * CH, CH)])

    def onesfill(i, _):
        buf[i, pl.ds(0, 16)] = o16
        return 0
    lax.fori_loop(0, CH, onesfill, 0)
    plsc.subcore_barrier()

    pltpu.sync_copy(dst3d.at[w], dst_st)

    def step(ch, _):
        pltpu.sync_copy(buf, cacc.at[dst_st.at[ch]], add=True)
        return 0
    lax.fori_loop(0, SEG_CPW, step, 0)
    plsc.subcore_barrier()

    for t in range(ACC_T):
        ck = s + NS * t

        @pl.when(ck < ACC_CH)
        def _():
            pltpu.sync_copy(cacc.at[pl.ds(ck * CH, CH)], buf)
            pltpu.sync_copy(buf, cntp.at[c, pl.ds(ck * CH, CH)])


@functools.partial(
    pl.kernel,
    out_type=jax.ShapeDtypeStruct((NW * DEC_CPW * CH // 8, D), jnp.float32),
    mesh=_mesh,
    scratch_types=[
        pltpu.VMEM((DEC_CPW, CH), jnp.int32),
        pltpu.VMEM((DEC_CPW, CH), jnp.int32),
        pltpu.VMEM((CH, D), jnp.float32),
        pltpu.VMEM((CH, D), jnp.float32),
        pltpu.VMEM((CH, D), jnp.float32),
        pltpu.VMEM((CH, D), jnp.float32),
        pltpu.VMEM((CH // 8, D), jnp.float32),
        pltpu.VMEM((1, D), jnp.float32),
        pltpu.SemaphoreType.DMA,
        pltpu.SemaphoreType.DMA,
        pltpu.SemaphoreType.DMA,
        pltpu.SemaphoreType.DMA,
    ],
)
def _dec_kernel(Ah, Bh, u3d, v3d, w2h, out, u_st, v_st, ra0, rb0, ra1, rb1,
                outstage, w2s, a0, b0, a1, b1):
    c = lax.axis_index("c")
    s = lax.axis_index("s")
    w = c * NS + s
    rbase = w * (DEC_CPW * CH // 8)
    pltpu.sync_copy(u3d.at[w], u_st)
    pltpu.sync_copy(v3d.at[w], v_st)
    pltpu.sync_copy(w2h, w2s)
    w2c = [w2s[0, pl.ds(j * 16, 16)] for j in range(D // 16)]

    RA = (ra0, ra1)
    RB = (rb0, rb1)
    SA = (a0, a1)
    SB = (b0, b1)

    def fire(lc, p):
        pltpu.async_copy(Ah.at[u_st.at[lc]], RA[p], SA[p])
        pltpu.async_copy(Bh.at[v_st.at[lc]], RB[p], SB[p])

    def wait(lc, p):
        pltpu.make_async_copy(Ah.at[u_st.at[lc]], RA[p], SA[p]).wait()
        pltpu.make_async_copy(Bh.at[v_st.at[lc]], RB[p], SB[p]).wait()

    def compute(ch, p):
        ra, rb = RA[p], RB[p]

        @plsc.parallel_loop(0, CH // 8, unroll=4)
        def _(row):
            for k in range(8):
                e = row * 8 + k
                acc0 = None
                acc1 = None
                for j in range(D // 16):
                    t = jnp.maximum(ra[e, pl.ds(j * 16, 16)] + rb[e, pl.ds(j * 16, 16)], 0.0)
                    term = t * w2c[j]
                    if j % 2 == 0:
                        acc0 = term if acc0 is None else acc0 + term
                    else:
                        acc1 = term if acc1 is None else acc1 + term
                outstage[row, pl.ds(k * 16, 16)] = acc0 + acc1
        pltpu.sync_copy(outstage, out.at[pl.ds(rbase + ch * (CH // 8), CH // 8)])

    fire(0, 0)
    fire(1, 1)

    def step(i, _):
        for p in range(2):
            lc = 2 * i + p
            wait(lc, p)
            compute(lc, p)
            nxt = lc + 2

            @pl.when(nxt < DEC_CPW)
            def _():
                fire(nxt, p)
        return 0
    lax.fori_loop(0, DEC_CPW // 2, step, 0)


_RED_BLK = 8192


def _tc_reduce(P, b2):
    def body(p_ref, b2_ref, o_ref):
        pv = p_ref[...]
        cols = [jnp.sum(pv[:, 16 * k:16 * (k + 1)], axis=1, keepdims=True)
                for k in range(8)]
        o_ref[...] = jnp.concatenate(cols, axis=1) + b2_ref[...]

    tot = P.shape[0]
    return pl.pallas_call(
        body,
        grid=(tot // _RED_BLK,),
        in_specs=[
            pl.BlockSpec((_RED_BLK, D), lambda i: (i, 0)),
            pl.BlockSpec((1, 8), lambda i: (0, 0)),
        ],
        out_specs=pl.BlockSpec((_RED_BLK, 8), lambda i: (i, 0)),
        out_shape=jax.ShapeDtypeStruct((tot, 8), jnp.float32),
    )(P, b2)

_ROWS_BLK = 1000


def _tc_layer1(x, aa, ab, ca, cb, wl, wr, b):
    def body(x_ref, aa_ref, ab_ref, ca_ref, cb_ref, wl_ref, wr_ref, b_ref, o_ref):
        cnt = ca_ref[:, 0:1] + cb_ref[:, 0:1]
        inv = 1.0 / jnp.maximum(cnt, 1.0)
        mean = (aa_ref[...] + ab_ref[...]) * inv
        acc = jnp.dot(mean, wl_ref[...], preferred_element_type=jnp.float32)
        acc = acc + jnp.dot(x_ref[...], wr_ref[...], preferred_element_type=jnp.float32)
        o_ref[...] = jnp.maximum(acc + b_ref[...], 0.0)

    return pl.pallas_call(
        body,
        grid=(N_NODES // _ROWS_BLK,),
        in_specs=[
            pl.BlockSpec((_ROWS_BLK, D), lambda i: (i, 0)),
            pl.BlockSpec((_ROWS_BLK, D), lambda i: (i, 0)),
            pl.BlockSpec((_ROWS_BLK, D), lambda i: (i, 0)),
            pl.BlockSpec((_ROWS_BLK, D), lambda i: (i, 0)),
            pl.BlockSpec((_ROWS_BLK, D), lambda i: (i, 0)),
            pl.BlockSpec((D, D), lambda i: (0, 0)),
            pl.BlockSpec((D, D), lambda i: (0, 0)),
            pl.BlockSpec((1, D), lambda i: (0, 0)),
        ],
        out_specs=pl.BlockSpec((_ROWS_BLK, D), lambda i: (i, 0)),
        out_shape=jax.ShapeDtypeStruct((N_NODES, D), jnp.float32),
    )(x, aa, ab, ca, cb, wl, wr, b)


def _tc_layer2(h, aa, ab, ca, cb, wl, wr, bl, wa, wb, bd):
    def body(h_ref, aa_ref, ab_ref, ca_ref, cb_ref, wl_ref, wr_ref, bl_ref,
             wa_ref, wb_ref, bd_ref, a_ref, b_ref):
        cnt = ca_ref[:, 0:1] + cb_ref[:, 0:1]
        inv = 1.0 / jnp.maximum(cnt, 1.0)
        mean = (aa_ref[...] + ab_ref[...]) * inv
        z = jnp.dot(mean, wl_ref[...], preferred_element_type=jnp.float32)
        z = z + jnp.dot(h_ref[...], wr_ref[...], preferred_element_type=jnp.float32)
        z = z + bl_ref[...]
        a_ref[...] = jnp.dot(z, wa_ref[...], preferred_element_type=jnp.float32) + bd_ref[...]
        b_ref[...] = jnp.dot(z, wb_ref[...], preferred_element_type=jnp.float32)

    return pl.pallas_call(
        body,
        grid=(N_NODES // _ROWS_BLK,),
        in_specs=[
            pl.BlockSpec((_ROWS_BLK, D), lambda i: (i, 0)),
            pl.BlockSpec((_ROWS_BLK, D), lambda i: (i, 0)),
            pl.BlockSpec((_ROWS_BLK, D), lambda i: (i, 0)),
            pl.BlockSpec((_ROWS_BLK, D), lambda i: (i, 0)),
            pl.BlockSpec((_ROWS_BLK, D), lambda i: (i, 0)),
            pl.BlockSpec((D, D), lambda i: (0, 0)),
            pl.BlockSpec((D, D), lambda i: (0, 0)),
            pl.BlockSpec((1, D), lambda i: (0, 0)),
            pl.BlockSpec((D, D), lambda i: (0, 0)),
            pl.BlockSpec((D, D), lambda i: (0, 0)),
            pl.BlockSpec((1, D), lambda i: (0, 0)),
        ],
        out_specs=[
            pl.BlockSpec((_ROWS_BLK, D), lambda i: (i, 0)),
            pl.BlockSpec((_ROWS_BLK, D), lambda i: (i, 0)),
        ],
        out_shape=[
            jax.ShapeDtypeStruct((N_NODES, D), jnp.float32),
            jax.ShapeDtypeStruct((N_NODES, D), jnp.float32),
        ],
    )(h, aa, ab, ca, cb, wl, wr, bl, wa, wb, bd)


def _pad_worker_idx(idx, epw, cpw, fill):
    idx = idx.astype(jnp.int32).reshape(NW, epw)
    pad = jnp.full((NW, cpw * CH - epw), fill, jnp.int32)
    return jnp.concatenate([idx, pad], axis=1).reshape(NW, cpw, CH)


def kernel(x, edge_index, pos_edge, neg_edge, Wl1, bl1, Wr1, Wl2, bl2, Wr2,
           Wd1, bd1, Wd2, bd2):
    src3d = _pad_worker_idx(edge_index[0], EPW, SEG_CPW, 0)
    dst3d = _pad_worker_idx(edge_index[1], EPW, SEG_CPW, PAD_NODE)
    u3d = _pad_worker_idx(jnp.concatenate([pos_edge[0], neg_edge[0]]),
                          DEC_EPW, DEC_CPW, 0)
    v3d = _pad_worker_idx(jnp.concatenate([pos_edge[1], neg_edge[1]]),
                          DEC_EPW, DEC_CPW, 0)

    cntp = _cnt_kernel(dst3d)
    ca = cntp[0, :N_NODES]
    cb = cntp[1, :N_NODES]
    aggp1 = _seg_kernel(x, src3d, dst3d)
    h = _tc_layer1(x, aggp1[0, :N_NODES], aggp1[1, :N_NODES], ca, cb,
                   Wl1.T, Wr1.T, bl1[None, :])
    aggp2 = _seg_kernel(h, src3d, dst3d)
    A, B = _tc_layer2(h, aggp2[0, :N_NODES], aggp2[1, :N_NODES], ca, cb,
                      Wl2.T, Wr2.T, bl2[None, :],
                      Wd1[:, :D].T, Wd1[:, D:].T, bd1[None, :])
    w2b = Wd2.reshape(1, D)
    lane_p = _dec_kernel(A, B, u3d, v3d, w2b)
    b2b = jnp.full((1, 8), bd2[0], jnp.float32)
    out = _tc_reduce(lane_p, b2b)
    return out.reshape(NW, DEC_CPW * CH)[:, :DEC_EPW].reshape(-1)
